# baseline (speedup 1.0000x reference)
import functools

import jax
import jax.numpy as jnp
from jax import lax
from jax.experimental import pallas as pl
from jax.experimental.pallas import tpu as pltpu
from jax.experimental.pallas import tpu_sc as plsc

N = 10000
E = 160000
D = 128
NC = 64

NTILES = 16
EPT = E // NTILES
CH = 128
KCH = 80
EPAD = KCH * CH - EPT
NA = N + 8
RPT = 624
RTAIL = N - RPT * NTILES


def _tile_ranges(s):
    r0 = pl.multiple_of(s * RPT, 8)
    tail0 = RPT * NTILES
    last = s == NTILES - 1
    return r0, tail0, last


NPH = 10
PH = KCH // NPH


def _agg_body(h_hbm, srcs_hbm, dsts_hbm, agg_out,
              srcA, dstA, srcB, dstB, rows0, rows1, agg_sh, sem0, sem1):
    c = lax.axis_index("c")
    s = lax.axis_index("s")
    r0, tail0, last = _tile_ranges(s)
    pltpu.sync_copy(h_hbm.at[pl.ds(r0, RPT)], agg_sh.at[pl.ds(r0, RPT)])

    @pl.when(last)
    def _init_tail():
        pltpu.sync_copy(h_hbm.at[pl.ds(tail0, RTAIL)],
                        agg_sh.at[pl.ds(tail0, RTAIL)])

    src_hbm_t = srcs_hbm.at[c].at[s]
    dst_hbm_t = dsts_hbm.at[c].at[s]
    pltpu.sync_copy(src_hbm_t.at[pl.ds(0, PH)], srcA)
    pltpu.sync_copy(dst_hbm_t.at[pl.ds(0, PH)], dstA)
    pltpu.sync_copy(src_hbm_t.at[pl.ds(PH, PH)], srcB)
    pltpu.sync_copy(dst_hbm_t.at[pl.ds(PH, PH)], dstB)
    plsc.subcore_barrier()

    pltpu.async_copy(h_hbm.at[srcA.at[0]], rows0, sem0)
    pltpu.async_copy(h_hbm.at[srcA.at[1]], rows1, sem1)
    bufs = (rows0, rows1)
    sems = (sem0, sem1)

    def phase(p, src_mine, dst_mine, src_oth):
        for l in range(PH):
            buf = bufs[l % 2]
            sem = sems[l % 2]
            pltpu.make_async_copy(h_hbm.at[src_mine.at[l]], buf, sem).wait()
            pltpu.sync_copy(buf, agg_sh.at[dst_mine.at[l]], add=True)
            if l + 2 < PH:
                pltpu.async_copy(h_hbm.at[src_mine.at[l + 2]], buf, sem)
            else:
                @pl.when(p < NPH - 1)
                def _prefetch_next_phase(l=l, buf=buf, sem=sem):
                    pltpu.async_copy(h_hbm.at[src_oth.at[l + 2 - PH]],
                                     buf, sem)
        @pl.when(p + 2 < NPH)
        def _restage():
            off = pl.multiple_of((p + 2) * PH, PH)
            pltpu.sync_copy(src_hbm_t.at[pl.ds(off, PH)], src_mine)
            pltpu.sync_copy(dst_hbm_t.at[pl.ds(off, PH)], dst_mine)

    def qstep(q, carry):
        phase(2 * q, srcA, dstA, srcB)
        phase(2 * q + 1, srcB, dstB, srcA)
        return carry

    lax.fori_loop(0, NPH // 2, qstep, 0)
    plsc.subcore_barrier()
    pltpu.sync_copy(agg_sh.at[pl.ds(r0, RPT)],
                    agg_out.at[c].at[pl.ds(r0, RPT)])

    @pl.when(last)
    def _out_tail():
        pltpu.sync_copy(agg_sh.at[pl.ds(tail0, RTAIL)],
                        agg_out.at[c].at[pl.ds(tail0, RTAIL)])


_agg = pl.kernel(
    _agg_body,
    out_type=jax.ShapeDtypeStruct((2, N, D), jnp.float32),
    mesh=plsc.VectorSubcoreMesh(core_axis_name="c", subcore_axis_name="s"),
    scratch_types=[
        pltpu.VMEM((PH, CH), jnp.int32),
        pltpu.VMEM((PH, CH), jnp.int32),
        pltpu.VMEM((PH, CH), jnp.int32),
        pltpu.VMEM((PH, CH), jnp.int32),
        pltpu.VMEM((CH, D), jnp.float32),
        pltpu.VMEM((CH, D), jnp.float32),
        pltpu.VMEM_SHARED((NA, D), jnp.float32),
        pltpu.SemaphoreType.DMA,
        pltpu.SemaphoreType.DMA,
    ],
)


def _deg_body(ones_hbm, dsts_hbm, deg_out, dst_v, ones_v, deg_sh):
    c = lax.axis_index("c")
    s = lax.axis_index("s")
    r0, tail0, last = _tile_ranges(s)
    pltpu.sync_copy(ones_hbm.at[pl.ds(r0, RPT)], deg_sh.at[pl.ds(r0, RPT)])
    pltpu.sync_copy(ones_hbm.at[pl.ds(0, CH)], ones_v)

    @pl.when(last)
    def _init_tail():
        pltpu.sync_copy(ones_hbm.at[pl.ds(tail0, RTAIL)],
                        deg_sh.at[pl.ds(tail0, RTAIL)])

    pltpu.sync_copy(dsts_hbm.at[c].at[s], dst_v)
    plsc.subcore_barrier()

    def step(j, carry):
        pltpu.sync_copy(ones_v, deg_sh.at[dst_v.at[j]], add=True)
        return carry

    lax.fori_loop(0, KCH, step, 0)
    plsc.subcore_barrier()
    pltpu.sync_copy(deg_sh.at[pl.ds(r0, RPT)],
                    deg_out.at[c].at[pl.ds(r0, RPT)])

    @pl.when(last)
    def _out_tail():
        pltpu.sync_copy(deg_sh.at[pl.ds(tail0, RTAIL)],
                        deg_out.at[c].at[pl.ds(tail0, RTAIL)])


_deg = pl.kernel(
    _deg_body,
    out_type=jax.ShapeDtypeStruct((2, N, D), jnp.float32),
    mesh=plsc.VectorSubcoreMesh(core_axis_name="c", subcore_axis_name="s"),
    scratch_types=[
        pltpu.VMEM((KCH, CH), jnp.int32),
        pltpu.VMEM((CH, D), jnp.float32),
        pltpu.VMEM_SHARED((NA, D), jnp.float32),
    ],
)

_RB = 2000


def _combine1_body(h_ref, a0_ref, a1_ref, d0_ref, d1_ref,
                   w0_ref, w1_ref, b0_ref, b1_ref, out_ref):
    inv0 = 1.0 / d0_ref[:, 0:1]
    inv1 = 1.0 / d1_ref[:, 0:1]
    hn0 = a0_ref[...] * inv0
    hn1 = a1_ref[...] * inv1
    acc = (jnp.dot(hn0, w0_ref[...], preferred_element_type=jnp.float32)
           + jnp.dot(hn1, w1_ref[...], preferred_element_type=jnp.float32)
           + b0_ref[...] + b1_ref[...])
    out_ref[...] = jnp.maximum(acc * 0.5, 0.0)


def _combine2_body(h_ref, a0_ref, a1_ref, d0_ref, d1_ref,
                   w0_ref, w1_ref, b0_ref, b1_ref, wc_ref, bc_ref, out_ref):
    inv0 = 1.0 / d0_ref[:, 0:1]
    inv1 = 1.0 / d1_ref[:, 0:1]
    hn0 = a0_ref[...] * inv0
    hn1 = a1_ref[...] * inv1
    acc = (jnp.dot(hn0, w0_ref[...], preferred_element_type=jnp.float32)
           + jnp.dot(hn1, w1_ref[...], preferred_element_type=jnp.float32)
           + b0_ref[...] + b1_ref[...]) * 0.5
    out_ref[...] = (jnp.dot(acc, wc_ref[...], preferred_element_type=jnp.float32)
                    + bc_ref[...])


def _row_specs():
    row = pl.BlockSpec((_RB, D), lambda i: (i, 0))
    deg = pl.BlockSpec((_RB, D), lambda i: (i, 0))
    w = pl.BlockSpec((D, D), lambda i: (0, 0))
    b = pl.BlockSpec((1, D), lambda i: (0, 0))
    return row, deg, w, b


def _combine1(h, a0, a1, d0, d1, w0, w1, b0, b1):
    row, deg, w, b = _row_specs()
    return pl.pallas_call(
        _combine1_body,
        grid=(N // _RB,),
        in_specs=[row, row, row, deg, deg, w, w, b, b],
        out_specs=row,
        out_shape=jax.ShapeDtypeStruct((N, D), jnp.float32),
    )(h, a0, a1, d0, d1, w0, w1, b0.reshape(1, D), b1.reshape(1, D))


def _combine2(h, a0, a1, d0, d1, w0, w1, b0, b1, wc, bc):
    row, deg, w, b = _row_specs()
    wcls = pl.BlockSpec((D, NC), lambda i: (0, 0))
    bcls = pl.BlockSpec((1, NC), lambda i: (0, 0))
    return pl.pallas_call(
        _combine2_body,
        grid=(N // _RB,),
        in_specs=[row, row, row, deg, deg, w, w, b, b, wcls, bcls],
        out_specs=pl.BlockSpec((_RB, NC), lambda i: (i, 0)),
        out_shape=jax.ShapeDtypeStruct((N, NC), jnp.float32),
    )(h, a0, a1, d0, d1, w0, w1, b0.reshape(1, D), b1.reshape(1, D),
      wc, bc.reshape(1, NC))


def _pad_edges(ei, fill):
    pad = jnp.full((NTILES, EPAD), fill, jnp.int32)
    return jnp.concatenate([ei, pad], axis=1).reshape(NTILES, KCH, CH)


def _sorted_tile_edges(edge_index):
    src = edge_index[0].reshape(NTILES, EPT)
    dst = edge_index[1].reshape(NTILES, EPT)
    order = jnp.argsort(src, axis=1)
    return (jnp.take_along_axis(src, order, axis=1),
            jnp.take_along_axis(dst, order, axis=1))


def kernel(x, edge_index_e0, edge_index_e1,
           W_0_e0, b_0_e0, W_0_e1, b_0_e1,
           W_1_e0, b_1_e0, W_1_e1, b_1_e1,
           W_cls, b_cls):
    src0, dst0 = _sorted_tile_edges(edge_index_e0)
    src1, dst1 = _sorted_tile_edges(edge_index_e1)
    srcs = jnp.stack([_pad_edges(src0, 0), _pad_edges(src1, 0)])
    dsts = jnp.stack([_pad_edges(dst0, N), _pad_edges(dst1, N)])
    ones = jnp.ones((N, D), jnp.float32)

    deg = _deg(ones, dsts)
    x_dep = jnp.where(deg[0][:, 0:1] > 0.0, x, 0.0)
    agg_x = _agg(x_dep, srcs, dsts)
    h1 = _combine1(x, agg_x[0], agg_x[1], deg[0], deg[1],
                   W_0_e0, W_0_e1, b_0_e0, b_0_e1)
    agg_h = _agg(h1, srcs, dsts)
    logits = _combine2(h1, agg_h[0], agg_h[1], deg[0], deg[1],
                       W_1_e0, W_1_e1, b_1_e0, b_1_e1, W_cls, b_cls)
    return logits


# speedup vs baseline: 1.4363x; 1.4363x over previous
import functools

import jax
import jax.numpy as jnp
from jax import lax
from jax.experimental import pallas as pl
from jax.experimental.pallas import tpu as pltpu
from jax.experimental.pallas import tpu_sc as plsc

N = 10000
E = 160000
D = 128
NC = 64

NTILES = 16
EPT = E // NTILES
CH = 128
KCH = 80
EPAD = KCH * CH - EPT
NA = N + 8
RPT = 624
RTAIL = N - RPT * NTILES


def _tile_ranges(s):
    r0 = pl.multiple_of(s * RPT, 8)
    tail0 = RPT * NTILES
    last = s == NTILES - 1
    return r0, tail0, last


NPH = 10
PH = KCH // NPH


def _agg_body(h_hbm, srcs_hbm, dsts_hbm, agg_out,
              srcA, dstA, srcB, dstB, rows0, rows1, agg_sh,
              sem0, sem1, rsem):
    c = lax.axis_index("c")
    s = lax.axis_index("s")
    r0, tail0, last = _tile_ranges(s)
    src_hbm_t = srcs_hbm.at[c].at[s]
    dst_hbm_t = dsts_hbm.at[c].at[s]
    pltpu.sync_copy(src_hbm_t.at[pl.ds(0, PH)], srcA)
    pltpu.sync_copy(dst_hbm_t.at[pl.ds(0, PH)], dstA)
    pltpu.async_copy(h_hbm.at[srcA.at[0]], rows0, sem0)
    pltpu.async_copy(h_hbm.at[srcA.at[1]], rows1, sem1)
    pltpu.sync_copy(src_hbm_t.at[pl.ds(PH, PH)], srcB)
    pltpu.sync_copy(dst_hbm_t.at[pl.ds(PH, PH)], dstB)

    pltpu.sync_copy(h_hbm.at[pl.ds(r0, RPT)], agg_sh.at[pl.ds(r0, RPT)])

    @pl.when(last)
    def _init_tail():
        pltpu.sync_copy(h_hbm.at[pl.ds(tail0, RTAIL)],
                        agg_sh.at[pl.ds(tail0, RTAIL)])

    plsc.subcore_barrier()

    bufs = (rows0, rows1)
    sems = (sem0, sem1)

    def phase(p, src_mine, dst_mine, src_oth, dst_oth):
        for l in range(PH):
            buf = bufs[l % 2]
            sem = sems[l % 2]
            if l == PH - 2:
                @pl.when((p >= 1) & (p < NPH - 1))
                def _await_restage():
                    pltpu.make_async_copy(
                        src_hbm_t.at[pl.ds(0, PH)], src_oth, rsem).wait()
                    pltpu.make_async_copy(
                        dst_hbm_t.at[pl.ds(0, PH)], dst_oth, rsem).wait()
            pltpu.make_async_copy(h_hbm.at[src_mine.at[l]], buf, sem).wait()
            pltpu.sync_copy(buf, agg_sh.at[dst_mine.at[l]], add=True)
            if l + 2 < PH:
                pltpu.async_copy(h_hbm.at[src_mine.at[l + 2]], buf, sem)
            else:
                @pl.when(p < NPH - 1)
                def _prefetch_next_phase(l=l, buf=buf, sem=sem):
                    pltpu.async_copy(h_hbm.at[src_oth.at[l + 2 - PH]],
                                     buf, sem)
        @pl.when(p + 2 < NPH)
        def _restage():
            off = pl.multiple_of((p + 2) * PH, PH)
            pltpu.async_copy(src_hbm_t.at[pl.ds(off, PH)], src_mine, rsem)
            pltpu.async_copy(dst_hbm_t.at[pl.ds(off, PH)], dst_mine, rsem)

    def qstep(q, carry):
        phase(2 * q, srcA, dstA, srcB, dstB)
        phase(2 * q + 1, srcB, dstB, srcA, dstA)
        return carry

    lax.fori_loop(0, NPH // 2, qstep, 0)
    plsc.subcore_barrier()
    pltpu.sync_copy(agg_sh.at[pl.ds(r0, RPT)],
                    agg_out.at[c].at[pl.ds(r0, RPT)])

    @pl.when(last)
    def _out_tail():
        pltpu.sync_copy(agg_sh.at[pl.ds(tail0, RTAIL)],
                        agg_out.at[c].at[pl.ds(tail0, RTAIL)])


_agg = pl.kernel(
    _agg_body,
    out_type=jax.ShapeDtypeStruct((2, N, D), jnp.float32),
    mesh=plsc.VectorSubcoreMesh(core_axis_name="c", subcore_axis_name="s"),
    scratch_types=[
        pltpu.VMEM((PH, CH), jnp.int32),
        pltpu.VMEM((PH, CH), jnp.int32),
        pltpu.VMEM((PH, CH), jnp.int32),
        pltpu.VMEM((PH, CH), jnp.int32),
        pltpu.VMEM((CH, D), jnp.float32),
        pltpu.VMEM((CH, D), jnp.float32),
        pltpu.VMEM_SHARED((NA, D), jnp.float32),
        pltpu.SemaphoreType.DMA,
        pltpu.SemaphoreType.DMA,
        pltpu.SemaphoreType.DMA,
    ],
)


def _deg_body(ones_hbm, dsts_hbm, deg_out, dst_v, ones_v, deg_sh):
    c = lax.axis_index("c")
    s = lax.axis_index("s")
    r0, tail0, last = _tile_ranges(s)
    pltpu.sync_copy(ones_hbm.at[pl.ds(r0, RPT)], deg_sh.at[pl.ds(r0, RPT)])
    pltpu.sync_copy(ones_hbm.at[pl.ds(0, CH)], ones_v)

    @pl.when(last)
    def _init_tail():
        pltpu.sync_copy(ones_hbm.at[pl.ds(tail0, RTAIL)],
                        deg_sh.at[pl.ds(tail0, RTAIL)])

    pltpu.sync_copy(dsts_hbm.at[c].at[s], dst_v)
    plsc.subcore_barrier()

    def step(j, carry):
        pltpu.sync_copy(ones_v, deg_sh.at[dst_v.at[j]], add=True)
        return carry

    lax.fori_loop(0, KCH, step, 0)
    plsc.subcore_barrier()
    pltpu.sync_copy(deg_sh.at[pl.ds(r0, RPT)],
                    deg_out.at[c].at[pl.ds(r0, RPT)])

    @pl.when(last)
    def _out_tail():
        pltpu.sync_copy(deg_sh.at[pl.ds(tail0, RTAIL)],
                        deg_out.at[c].at[pl.ds(tail0, RTAIL)])


_deg = pl.kernel(
    _deg_body,
    out_type=jax.ShapeDtypeStruct((2, N, D), jnp.float32),
    mesh=plsc.VectorSubcoreMesh(core_axis_name="c", subcore_axis_name="s"),
    scratch_types=[
        pltpu.VMEM((KCH, CH), jnp.int32),
        pltpu.VMEM((CH, D), jnp.float32),
        pltpu.VMEM_SHARED((NA, D), jnp.float32),
    ],
)

_RB = 2000


def _combine1_body(h_ref, a0_ref, a1_ref, d0_ref, d1_ref,
                   w0_ref, w1_ref, b0_ref, b1_ref, out_ref):
    inv0 = 1.0 / d0_ref[:, 0:1]
    inv1 = 1.0 / d1_ref[:, 0:1]
    hn0 = a0_ref[...] * inv0
    hn1 = a1_ref[...] * inv1
    acc = (jnp.dot(hn0, w0_ref[...], preferred_element_type=jnp.float32)
           + jnp.dot(hn1, w1_ref[...], preferred_element_type=jnp.float32)
           + b0_ref[...] + b1_ref[...])
    out_ref[...] = jnp.maximum(acc * 0.5, 0.0)


def _combine2_body(h_ref, a0_ref, a1_ref, d0_ref, d1_ref,
                   w0_ref, w1_ref, b0_ref, b1_ref, wc_ref, bc_ref, out_ref):
    inv0 = 1.0 / d0_ref[:, 0:1]
    inv1 = 1.0 / d1_ref[:, 0:1]
    hn0 = a0_ref[...] * inv0
    hn1 = a1_ref[...] * inv1
    acc = (jnp.dot(hn0, w0_ref[...], preferred_element_type=jnp.float32)
           + jnp.dot(hn1, w1_ref[...], preferred_element_type=jnp.float32)
           + b0_ref[...] + b1_ref[...]) * 0.5
    out_ref[...] = (jnp.dot(acc, wc_ref[...], preferred_element_type=jnp.float32)
                    + bc_ref[...])


def _row_specs():
    row = pl.BlockSpec((_RB, D), lambda i: (i, 0))
    deg = pl.BlockSpec((_RB, D), lambda i: (i, 0))
    w = pl.BlockSpec((D, D), lambda i: (0, 0))
    b = pl.BlockSpec((1, D), lambda i: (0, 0))
    return row, deg, w, b


def _combine1(h, a0, a1, d0, d1, w0, w1, b0, b1):
    row, deg, w, b = _row_specs()
    return pl.pallas_call(
        _combine1_body,
        grid=(N // _RB,),
        in_specs=[row, row, row, deg, deg, w, w, b, b],
        out_specs=row,
        out_shape=jax.ShapeDtypeStruct((N, D), jnp.float32),
    )(h, a0, a1, d0, d1, w0, w1, b0.reshape(1, D), b1.reshape(1, D))


def _combine2(h, a0, a1, d0, d1, w0, w1, b0, b1, wc, bc):
    row, deg, w, b = _row_specs()
    wcls = pl.BlockSpec((D, NC), lambda i: (0, 0))
    bcls = pl.BlockSpec((1, NC), lambda i: (0, 0))
    return pl.pallas_call(
        _combine2_body,
        grid=(N // _RB,),
        in_specs=[row, row, row, deg, deg, w, w, b, b, wcls, bcls],
        out_specs=pl.BlockSpec((_RB, NC), lambda i: (i, 0)),
        out_shape=jax.ShapeDtypeStruct((N, NC), jnp.float32),
    )(h, a0, a1, d0, d1, w0, w1, b0.reshape(1, D), b1.reshape(1, D),
      wc, bc.reshape(1, NC))


def _pad_edges(ei, fill):
    per_tile = ei.reshape(NTILES, EPT)
    pad = jnp.full((NTILES, EPAD), fill, jnp.int32)
    return jnp.concatenate([per_tile, pad], axis=1).reshape(NTILES, KCH, CH)


def kernel(x, edge_index_e0, edge_index_e1,
           W_0_e0, b_0_e0, W_0_e1, b_0_e1,
           W_1_e0, b_1_e0, W_1_e1, b_1_e1,
           W_cls, b_cls):
    srcs = jnp.stack([_pad_edges(edge_index_e0[0], 0),
                      _pad_edges(edge_index_e1[0], 0)])
    dsts = jnp.stack([_pad_edges(edge_index_e0[1], N),
                      _pad_edges(edge_index_e1[1], N)])
    ones = jnp.ones((N, D), jnp.float32)

    deg = _deg(ones, dsts)
    x_dep = jnp.where(deg[0][:, 0:1] > 0.0, x, 0.0)
    agg_x = _agg(x_dep, srcs, dsts)
    h1 = _combine1(x, agg_x[0], agg_x[1], deg[0], deg[1],
                   W_0_e0, W_0_e1, b_0_e0, b_0_e1)
    agg_h = _agg(h1, srcs, dsts)
    logits = _combine2(h1, agg_h[0], agg_h[1], deg[0], deg[1],
                       W_1_e0, W_1_e1, b_1_e0, b_1_e1, W_cls, b_cls)
    return logits


# speedup vs baseline: 1.4407x; 1.0030x over previous
import functools

import jax
import jax.numpy as jnp
from jax import lax
from jax.experimental import pallas as pl
from jax.experimental.pallas import tpu as pltpu
from jax.experimental.pallas import tpu_sc as plsc

N = 10000
E = 160000
D = 128
NC = 64

NTILES = 16
EPT = E // NTILES
CH = 128
KCH = 80
EPAD = KCH * CH - EPT
NA = N + 8
RPT = 624
RTAIL = N - RPT * NTILES


def _tile_ranges(s):
    r0 = pl.multiple_of(s * RPT, 8)
    tail0 = RPT * NTILES
    last = s == NTILES - 1
    return r0, tail0, last


NPH = 10
PH = KCH // NPH


def _agg_body(h_hbm, srcs_hbm, dsts_hbm, agg_out,
              srcA, dstA, srcB, dstB, rows0, rows1, agg_sh,
              sem0, sem1, rsem):
    c = lax.axis_index("c")
    s = lax.axis_index("s")
    r0, tail0, last = _tile_ranges(s)
    src_hbm_t = srcs_hbm.at[c].at[s]
    dst_hbm_t = dsts_hbm.at[c].at[s]
    pltpu.sync_copy(src_hbm_t.at[pl.ds(0, PH)], srcA)
    pltpu.sync_copy(dst_hbm_t.at[pl.ds(0, PH)], dstA)
    pltpu.async_copy(h_hbm.at[srcA.at[0]], rows0, sem0)
    pltpu.async_copy(h_hbm.at[srcA.at[1]], rows1, sem1)
    pltpu.sync_copy(src_hbm_t.at[pl.ds(PH, PH)], srcB)
    pltpu.sync_copy(dst_hbm_t.at[pl.ds(PH, PH)], dstB)

    pltpu.sync_copy(h_hbm.at[pl.ds(r0, RPT)], agg_sh.at[pl.ds(r0, RPT)])

    @pl.when(last)
    def _init_tail():
        pltpu.sync_copy(h_hbm.at[pl.ds(tail0, RTAIL)],
                        agg_sh.at[pl.ds(tail0, RTAIL)])

    plsc.subcore_barrier()

    bufs = (rows0, rows1)
    sems = (sem0, sem1)

    def phase(p, src_mine, dst_mine, src_oth, dst_oth):
        for l in range(PH):
            buf = bufs[l % 2]
            sem = sems[l % 2]
            if l == PH - 2:
                @pl.when((p >= 1) & (p < NPH - 1))
                def _await_restage():
                    pltpu.make_async_copy(
                        src_hbm_t.at[pl.ds(0, PH)], src_oth, rsem).wait()
                    pltpu.make_async_copy(
                        dst_hbm_t.at[pl.ds(0, PH)], dst_oth, rsem).wait()
            pltpu.make_async_copy(h_hbm.at[src_mine.at[l]], buf, sem).wait()
            pltpu.sync_copy(buf, agg_sh.at[dst_mine.at[l]], add=True)
            if l + 2 < PH:
                pltpu.async_copy(h_hbm.at[src_mine.at[l + 2]], buf, sem)
            else:
                @pl.when(p < NPH - 1)
                def _prefetch_next_phase(l=l, buf=buf, sem=sem):
                    pltpu.async_copy(h_hbm.at[src_oth.at[l + 2 - PH]],
                                     buf, sem)
        @pl.when(p + 2 < NPH)
        def _restage():
            off = pl.multiple_of((p + 2) * PH, PH)
            pltpu.async_copy(src_hbm_t.at[pl.ds(off, PH)], src_mine, rsem)
            pltpu.async_copy(dst_hbm_t.at[pl.ds(off, PH)], dst_mine, rsem)

    def qstep(q, carry):
        phase(2 * q, srcA, dstA, srcB, dstB)
        phase(2 * q + 1, srcB, dstB, srcA, dstA)
        return carry

    lax.fori_loop(0, NPH // 2, qstep, 0)
    plsc.subcore_barrier()
    pltpu.sync_copy(agg_sh.at[pl.ds(r0, RPT)],
                    agg_out.at[c].at[pl.ds(r0, RPT)])

    @pl.when(last)
    def _out_tail():
        pltpu.sync_copy(agg_sh.at[pl.ds(tail0, RTAIL)],
                        agg_out.at[c].at[pl.ds(tail0, RTAIL)])


_agg = pl.kernel(
    _agg_body,
    out_type=jax.ShapeDtypeStruct((2, N, D), jnp.float32),
    mesh=plsc.VectorSubcoreMesh(core_axis_name="c", subcore_axis_name="s"),
    scratch_types=[
        pltpu.VMEM((PH, CH), jnp.int32),
        pltpu.VMEM((PH, CH), jnp.int32),
        pltpu.VMEM((PH, CH), jnp.int32),
        pltpu.VMEM((PH, CH), jnp.int32),
        pltpu.VMEM((CH, D), jnp.float32),
        pltpu.VMEM((CH, D), jnp.float32),
        pltpu.VMEM_SHARED((NA, D), jnp.float32),
        pltpu.SemaphoreType.DMA,
        pltpu.SemaphoreType.DMA,
        pltpu.SemaphoreType.DMA,
    ],
)


def _deg_body(ones_hbm, dsts_hbm, deg_out, dst_v, ones_v, deg_sh):
    c = lax.axis_index("c")
    s = lax.axis_index("s")
    r0, tail0, last = _tile_ranges(s)
    pltpu.sync_copy(ones_hbm.at[pl.ds(r0, RPT)], deg_sh.at[pl.ds(r0, RPT)])
    pltpu.sync_copy(ones_hbm.at[pl.ds(0, CH)], ones_v)

    @pl.when(last)
    def _init_tail():
        pltpu.sync_copy(ones_hbm.at[pl.ds(tail0, RTAIL)],
                        deg_sh.at[pl.ds(tail0, RTAIL)])

    pltpu.sync_copy(dsts_hbm.at[c].at[s], dst_v)
    plsc.subcore_barrier()

    def step(j, carry):
        pltpu.sync_copy(ones_v, deg_sh.at[dst_v.at[j]], add=True)
        return carry

    lax.fori_loop(0, KCH, step, 0)
    plsc.subcore_barrier()
    pltpu.sync_copy(deg_sh.at[pl.ds(r0, RPT)],
                    deg_out.at[c].at[pl.ds(r0, RPT)])

    @pl.when(last)
    def _out_tail():
        pltpu.sync_copy(deg_sh.at[pl.ds(tail0, RTAIL)],
                        deg_out.at[c].at[pl.ds(tail0, RTAIL)])


_deg = pl.kernel(
    _deg_body,
    out_type=jax.ShapeDtypeStruct((2, N, D), jnp.float32),
    mesh=plsc.VectorSubcoreMesh(core_axis_name="c", subcore_axis_name="s"),
    scratch_types=[
        pltpu.VMEM((KCH, CH), jnp.int32),
        pltpu.VMEM((CH, D), jnp.float32),
        pltpu.VMEM_SHARED((NA, D), jnp.float32),
    ],
)

_RB = 2000


def _mm_body(a0_ref, a1_ref, w0_ref, w1_ref, m0_ref, m1_ref):
    m0_ref[...] = jnp.dot(a0_ref[...], w0_ref[...],
                          preferred_element_type=jnp.float32)
    m1_ref[...] = jnp.dot(a1_ref[...], w1_ref[...],
                          preferred_element_type=jnp.float32)


def _fin1_body(m0_ref, m1_ref, d0_ref, d1_ref, b0_ref, b1_ref, out_ref):
    inv0 = 1.0 / d0_ref[:, 0:1]
    inv1 = 1.0 / d1_ref[:, 0:1]
    acc = (m0_ref[...] * inv0 + m1_ref[...] * inv1
           + b0_ref[...] + b1_ref[...])
    out_ref[...] = jnp.maximum(acc * 0.5, 0.0)


def _combine2_body(a0_ref, a1_ref, d0_ref, d1_ref,
                   w0_ref, w1_ref, b0_ref, b1_ref, wc_ref, bc_ref, out_ref):
    inv0 = 1.0 / d0_ref[:, 0:1]
    inv1 = 1.0 / d1_ref[:, 0:1]
    hn0 = a0_ref[...] * inv0
    hn1 = a1_ref[...] * inv1
    acc = (jnp.dot(hn0, w0_ref[...], preferred_element_type=jnp.float32)
           + jnp.dot(hn1, w1_ref[...], preferred_element_type=jnp.float32)
           + b0_ref[...] + b1_ref[...]) * 0.5
    out_ref[...] = (jnp.dot(acc, wc_ref[...], preferred_element_type=jnp.float32)
                    + bc_ref[...])


def _row_specs():
    row = pl.BlockSpec((_RB, D), lambda i: (i, 0))
    deg = pl.BlockSpec((_RB, D), lambda i: (i, 0))
    w = pl.BlockSpec((D, D), lambda i: (0, 0))
    b = pl.BlockSpec((1, D), lambda i: (0, 0))
    return row, deg, w, b


def _mm(a0, a1, w0, w1):
    row, _, w, _ = _row_specs()
    return pl.pallas_call(
        _mm_body,
        grid=(N // _RB,),
        in_specs=[row, row, w, w],
        out_specs=(row, row),
        out_shape=(jax.ShapeDtypeStruct((N, D), jnp.float32),
                   jax.ShapeDtypeStruct((N, D), jnp.float32)),
    )(a0, a1, w0, w1)


def _fin1(m0, m1, d0, d1, b0, b1):
    row, deg, _, b = _row_specs()
    return pl.pallas_call(
        _fin1_body,
        grid=(N // _RB,),
        in_specs=[row, row, deg, deg, b, b],
        out_specs=row,
        out_shape=jax.ShapeDtypeStruct((N, D), jnp.float32),
    )(m0, m1, d0, d1, b0.reshape(1, D), b1.reshape(1, D))


def _combine2(a0, a1, d0, d1, w0, w1, b0, b1, wc, bc):
    row, deg, w, b = _row_specs()
    wcls = pl.BlockSpec((D, NC), lambda i: (0, 0))
    bcls = pl.BlockSpec((1, NC), lambda i: (0, 0))
    return pl.pallas_call(
        _combine2_body,
        grid=(N // _RB,),
        in_specs=[row, row, deg, deg, w, w, b, b, wcls, bcls],
        out_specs=pl.BlockSpec((_RB, NC), lambda i: (i, 0)),
        out_shape=jax.ShapeDtypeStruct((N, NC), jnp.float32),
    )(a0, a1, d0, d1, w0, w1, b0.reshape(1, D), b1.reshape(1, D),
      wc, bc.reshape(1, NC))


def _pad_edges(ei, fill):
    per_tile = ei.reshape(NTILES, EPT)
    pad = jnp.full((NTILES, EPAD), fill, jnp.int32)
    return jnp.concatenate([per_tile, pad], axis=1).reshape(NTILES, KCH, CH)


def kernel(x, edge_index_e0, edge_index_e1,
           W_0_e0, b_0_e0, W_0_e1, b_0_e1,
           W_1_e0, b_1_e0, W_1_e1, b_1_e1,
           W_cls, b_cls):
    srcs = jnp.stack([_pad_edges(edge_index_e0[0], 0),
                      _pad_edges(edge_index_e1[0], 0)])
    dsts = jnp.stack([_pad_edges(edge_index_e0[1], N),
                      _pad_edges(edge_index_e1[1], N)])
    agg_x = _agg(x, srcs, dsts)
    ones = jnp.where(agg_x[0][:, 0:1] < jnp.inf, 1.0, 0.0) * jnp.ones((N, D),
                                                                      jnp.float32)
    m0, m1 = _mm(agg_x[0], agg_x[1], W_0_e0, W_0_e1)
    deg = _deg(ones, dsts)
    h1 = _fin1(m0, m1, deg[0], deg[1], b_0_e0, b_0_e1)
    agg_h = _agg(h1, srcs, dsts)
    logits = _combine2(agg_h[0], agg_h[1], deg[0], deg[1],
                       W_1_e0, W_1_e1, b_1_e0, b_1_e1, W_cls, b_cls)
    return logits
